# Initial kernel scaffold; baseline (speedup 1.0000x reference)
import jax
import jax.numpy as jnp
from jax.experimental import pallas as pl


def kernel(pred, target):
    raise NotImplementedError("write your pallas kernel here")


# speedup vs baseline: 7.7664x; 7.7664x over previous
import math

import jax
import jax.numpy as jnp
from jax.experimental import pallas as pl

_NUM_CLASSES = 722
_V1 = math.exp(-2.0 / 4.0)
_V2 = math.exp(-4.0 / 4.0)
_V3 = math.exp(-8.0 / 4.0)
_ROW_BLOCK = 512


def _loss_kernel(pred_ref, tgt_ref, out_ref):
    x = pred_ref[...]
    t = tgt_ref[...]

    m = jnp.max(x, axis=1, keepdims=True)
    s = jnp.sum(jnp.exp(x - m), axis=1, keepdims=True)
    lse = m + jnp.log(s)

    j = jax.lax.broadcasted_iota(jnp.int32, x.shape, 1)
    d = jnp.abs(j - t)
    w = jnp.where(d == 0, 1.0,
        jnp.where(d == 1, _V1,
        jnp.where(d == 2, _V2,
        jnp.where(d == 3, _V3, 0.0))))

    wsum = jnp.sum(w, axis=1, keepdims=True)
    wpred = jnp.sum(w * x, axis=1, keepdims=True)

    n_rows = pl.num_programs(0) * x.shape[0]
    partial = jnp.sum(wsum * lse - wpred, keepdims=True).reshape(1, 1) * (1.0 / n_rows)

    @pl.when(pl.program_id(0) == 0)
    def _():
        out_ref[...] = jnp.zeros_like(out_ref)

    out_ref[...] += partial


def kernel(pred, target):
    B, T, C = pred.shape
    n = B * T
    pred2 = pred.reshape(n, C)
    tgt2 = target.reshape(n, 1)
    grid = n // _ROW_BLOCK

    out = pl.pallas_call(
        _loss_kernel,
        grid=(grid,),
        in_specs=[
            pl.BlockSpec((_ROW_BLOCK, C), lambda i: (i, 0)),
            pl.BlockSpec((_ROW_BLOCK, 1), lambda i: (i, 0)),
        ],
        out_specs=pl.BlockSpec((1, 1), lambda i: (0, 0)),
        out_shape=jax.ShapeDtypeStruct((1, 1), jnp.float32),
    )(pred2, tgt2)
    return out[0, 0]
